# baseline (device time: 23245 ns/iter reference)
import jax
import jax.numpy as jnp
from jax import lax
from jax.experimental import pallas as pl
from jax.experimental.pallas import tpu as pltpu

M = 512
F = 2048
M_HALF = M // 2
F_HALF = F // 2


def kernel(x, dy):
    def body(x_ref, dy_ref, out_ref,
             send_x, recv_x, send_y, recv_y,
             send_sems, recv_sems):
        my_x = lax.axis_index("x")
        my_y = lax.axis_index("y")

        barrier_sem = pltpu.get_barrier_semaphore()
        pl.semaphore_signal(barrier_sem, inc=1, device_id=(1 - my_x, my_y),
                            device_id_type=pl.DeviceIdType.MESH)
        pl.semaphore_signal(barrier_sem, inc=1, device_id=(my_x, 1 - my_y),
                            device_id_type=pl.DeviceIdType.MESH)
        pl.semaphore_wait(barrier_sem, 2)

        xv = x_ref[...].astype(jnp.bfloat16)
        dyv = dy_ref[:, pl.ds(my_y * F_HALF, F_HALF)].astype(jnp.bfloat16)
        p = lax.dot_general(xv, dyv, (((0,), (0,)), ((), ())),
                            preferred_element_type=jnp.float32)

        top = p[:M_HALF, :]
        bot = p[M_HALF:, :]
        mine = jnp.where(my_x == 0, top, bot)
        theirs = jnp.where(my_x == 0, bot, top)
        send_x[...] = theirs.astype(jnp.bfloat16)

        rdma_x = pltpu.make_async_remote_copy(
            src_ref=send_x, dst_ref=recv_x,
            send_sem=send_sems.at[0], recv_sem=recv_sems.at[0],
            device_id=(1 - my_x, my_y), device_id_type=pl.DeviceIdType.MESH)
        rdma_x.start()
        rdma_x.wait()

        q = mine + recv_x[...].astype(jnp.float32)

        send_y[...] = q.astype(jnp.bfloat16)
        rdma_y = pltpu.make_async_remote_copy(
            src_ref=send_y, dst_ref=recv_y,
            send_sem=send_sems.at[1], recv_sem=recv_sems.at[1],
            device_id=(my_x, 1 - my_y), device_id_type=pl.DeviceIdType.MESH)
        rdma_y.start()
        rdma_y.wait()

        out_ref[:, pl.ds(my_y * F_HALF, F_HALF)] = q
        out_ref[:, pl.ds((1 - my_y) * F_HALF, F_HALF)] = (
            recv_y[...].astype(jnp.float32))

    return pl.pallas_call(
        body,
        out_shape=jax.ShapeDtypeStruct((M_HALF, F), jnp.float32),
        in_specs=[pl.BlockSpec(memory_space=pltpu.VMEM),
                  pl.BlockSpec(memory_space=pltpu.VMEM)],
        out_specs=pl.BlockSpec(memory_space=pltpu.VMEM),
        scratch_shapes=[
            pltpu.VMEM((M_HALF, F_HALF), jnp.bfloat16),
            pltpu.VMEM((M_HALF, F_HALF), jnp.bfloat16),
            pltpu.VMEM((M_HALF, F_HALF), jnp.bfloat16),
            pltpu.VMEM((M_HALF, F_HALF), jnp.bfloat16),
            pltpu.SemaphoreType.DMA((2,)),
            pltpu.SemaphoreType.DMA((2,)),
        ],
        compiler_params=pltpu.CompilerParams(collective_id=0),
    )(x, dy)


# device time: 19147 ns/iter; 1.2140x vs baseline; 1.2140x over previous
import jax
import jax.numpy as jnp
from jax import lax
from jax.experimental import pallas as pl
from jax.experimental.pallas import tpu as pltpu

M = 512
F = 2048
M_HALF = M // 2
F_HALF = F // 2
NC = 8
CS = F_HALF // NC


def kernel(x, dy):
    def body(x_ref, dy_ref, out_ref,
             send_x, recv_x, send_y, recv_y,
             sx_sems, rx_sems, sy_sems, ry_sems):
        my_x = lax.axis_index("x")
        my_y = lax.axis_index("y")

        barrier_sem = pltpu.get_barrier_semaphore()
        pl.semaphore_signal(barrier_sem, inc=1, device_id=(1 - my_x, my_y),
                            device_id_type=pl.DeviceIdType.MESH)
        pl.semaphore_signal(barrier_sem, inc=1, device_id=(my_x, 1 - my_y),
                            device_id_type=pl.DeviceIdType.MESH)
        pl.semaphore_wait(barrier_sem, 2)

        xv = x_ref[...].astype(jnp.bfloat16)

        mines = []
        x_rdmas = []
        for c in range(NC):
            dyc = dy_ref[:, pl.ds(my_y * F_HALF + c * CS, CS)].astype(
                jnp.bfloat16)
            p = lax.dot_general(xv, dyc, (((0,), (0,)), ((), ())),
                                preferred_element_type=jnp.float32)
            top = p[:M_HALF, :]
            bot = p[M_HALF:, :]
            mines.append(jnp.where(my_x == 0, top, bot))
            send_x[:, c * CS:(c + 1) * CS] = jnp.where(
                my_x == 0, bot, top).astype(jnp.bfloat16)
            rdma = pltpu.make_async_remote_copy(
                src_ref=send_x.at[:, pl.ds(c * CS, CS)],
                dst_ref=recv_x.at[:, pl.ds(c * CS, CS)],
                send_sem=sx_sems.at[c], recv_sem=rx_sems.at[c],
                device_id=(1 - my_x, my_y),
                device_id_type=pl.DeviceIdType.MESH)
            rdma.start()
            x_rdmas.append(rdma)

        y_rdmas = []
        for c in range(NC):
            x_rdmas[c].wait()
            q = mines[c] + recv_x[:, c * CS:(c + 1) * CS].astype(jnp.float32)
            send_y[:, c * CS:(c + 1) * CS] = q.astype(jnp.bfloat16)
            rdma = pltpu.make_async_remote_copy(
                src_ref=send_y.at[:, pl.ds(c * CS, CS)],
                dst_ref=recv_y.at[:, pl.ds(c * CS, CS)],
                send_sem=sy_sems.at[c], recv_sem=ry_sems.at[c],
                device_id=(my_x, 1 - my_y),
                device_id_type=pl.DeviceIdType.MESH)
            rdma.start()
            y_rdmas.append(rdma)
            out_ref[:, pl.ds(my_y * F_HALF + c * CS, CS)] = q

        for c in range(NC):
            y_rdmas[c].wait()
            out_ref[:, pl.ds((1 - my_y) * F_HALF + c * CS, CS)] = (
                recv_y[:, c * CS:(c + 1) * CS].astype(jnp.float32))

    return pl.pallas_call(
        body,
        out_shape=jax.ShapeDtypeStruct((M_HALF, F), jnp.float32),
        in_specs=[pl.BlockSpec(memory_space=pltpu.VMEM),
                  pl.BlockSpec(memory_space=pltpu.VMEM)],
        out_specs=pl.BlockSpec(memory_space=pltpu.VMEM),
        scratch_shapes=[
            pltpu.VMEM((M_HALF, F_HALF), jnp.bfloat16),
            pltpu.VMEM((M_HALF, F_HALF), jnp.bfloat16),
            pltpu.VMEM((M_HALF, F_HALF), jnp.bfloat16),
            pltpu.VMEM((M_HALF, F_HALF), jnp.bfloat16),
            pltpu.SemaphoreType.DMA((NC,)),
            pltpu.SemaphoreType.DMA((NC,)),
            pltpu.SemaphoreType.DMA((NC,)),
            pltpu.SemaphoreType.DMA((NC,)),
        ],
        compiler_params=pltpu.CompilerParams(collective_id=0),
    )(x, dy)


# device time: 18021 ns/iter; 1.2899x vs baseline; 1.0625x over previous
import jax
import jax.numpy as jnp
from jax import lax
from jax.experimental import pallas as pl
from jax.experimental.pallas import tpu as pltpu

M = 512
F = 2048
M_HALF = M // 2
F_HALF = F // 2
NC = 8
CS = F_HALF // NC


def kernel(x, dy):
    def body(x_ref, dy_ref, out_ref,
             send_x, recv_x, send_y, recv_y,
             sx_sems, rx_sems, sy_sems, ry_sems):
        my_x = lax.axis_index("x")
        my_y = lax.axis_index("y")

        barrier_sem = pltpu.get_barrier_semaphore()
        pl.semaphore_signal(barrier_sem, inc=1, device_id=(1 - my_x, my_y),
                            device_id_type=pl.DeviceIdType.MESH)
        pl.semaphore_signal(barrier_sem, inc=1, device_id=(my_x, 1 - my_y),
                            device_id_type=pl.DeviceIdType.MESH)

        xm = x_ref[:, pl.ds(my_x * M_HALF, M_HALF)]
        xt = x_ref[:, pl.ds((1 - my_x) * M_HALF, M_HALF)]

        mines = []
        x_rdmas = []
        for c in range(NC):
            dyc = dy_ref[:, pl.ds(my_y * F_HALF + c * CS, CS)]
            pt = lax.dot_general(xt, dyc, (((0,), (0,)), ((), ())),
                                 preferred_element_type=jnp.float32)
            send_x[:, c * CS:(c + 1) * CS] = pt.astype(jnp.bfloat16)
            if c == 0:
                pl.semaphore_wait(barrier_sem, 2)
            rdma = pltpu.make_async_remote_copy(
                src_ref=send_x.at[:, pl.ds(c * CS, CS)],
                dst_ref=recv_x.at[:, pl.ds(c * CS, CS)],
                send_sem=sx_sems.at[c], recv_sem=rx_sems.at[c],
                device_id=(1 - my_x, my_y),
                device_id_type=pl.DeviceIdType.MESH)
            rdma.start()
            x_rdmas.append(rdma)
            mines.append(lax.dot_general(xm, dyc, (((0,), (0,)), ((), ())),
                                         preferred_element_type=jnp.float32))

        y_rdmas = []
        for c in range(NC):
            x_rdmas[c].wait()
            q = mines[c] + recv_x[:, c * CS:(c + 1) * CS].astype(jnp.float32)
            send_y[:, c * CS:(c + 1) * CS] = q.astype(jnp.bfloat16)
            rdma = pltpu.make_async_remote_copy(
                src_ref=send_y.at[:, pl.ds(c * CS, CS)],
                dst_ref=recv_y.at[:, pl.ds(c * CS, CS)],
                send_sem=sy_sems.at[c], recv_sem=ry_sems.at[c],
                device_id=(my_x, 1 - my_y),
                device_id_type=pl.DeviceIdType.MESH)
            rdma.start()
            y_rdmas.append(rdma)
            out_ref[:, pl.ds(my_y * F_HALF + c * CS, CS)] = q

        for c in range(NC):
            y_rdmas[c].wait()
            out_ref[:, pl.ds((1 - my_y) * F_HALF + c * CS, CS)] = (
                recv_y[:, c * CS:(c + 1) * CS].astype(jnp.float32))

    return pl.pallas_call(
        body,
        out_shape=jax.ShapeDtypeStruct((M_HALF, F), jnp.float32),
        in_specs=[pl.BlockSpec(memory_space=pltpu.VMEM),
                  pl.BlockSpec(memory_space=pltpu.VMEM)],
        out_specs=pl.BlockSpec(memory_space=pltpu.VMEM),
        scratch_shapes=[
            pltpu.VMEM((M_HALF, F_HALF), jnp.bfloat16),
            pltpu.VMEM((M_HALF, F_HALF), jnp.bfloat16),
            pltpu.VMEM((M_HALF, F_HALF), jnp.bfloat16),
            pltpu.VMEM((M_HALF, F_HALF), jnp.bfloat16),
            pltpu.SemaphoreType.DMA((NC,)),
            pltpu.SemaphoreType.DMA((NC,)),
            pltpu.SemaphoreType.DMA((NC,)),
            pltpu.SemaphoreType.DMA((NC,)),
        ],
        compiler_params=pltpu.CompilerParams(collective_id=0),
    )(x, dy)


# device time: 17909 ns/iter; 1.2980x vs baseline; 1.0063x over previous
import jax
import jax.numpy as jnp
from jax import lax
from jax.experimental import pallas as pl
from jax.experimental.pallas import tpu as pltpu

M = 512
F = 2048
M_HALF = M // 2
F_HALF = F // 2
NC = 8
CS = F_HALF // NC


def kernel(x, dy):
    def body(x_ref, dy_ref, out_ref,
             send_x, recv_x, send_y, recv_y,
             sx_sems, rx_sems, sy_sems, ry_sems):
        my_x = lax.axis_index("x")
        my_y = lax.axis_index("y")

        def inner(ysem):
            bs = pltpu.get_barrier_semaphore()
            pl.semaphore_signal(bs, inc=1, device_id=(1 - my_x, my_y),
                                device_id_type=pl.DeviceIdType.MESH)
            pl.semaphore_signal(ysem, inc=1, device_id=(my_x, 1 - my_y),
                                device_id_type=pl.DeviceIdType.MESH)

            xm = x_ref[:, pl.ds(my_x * M_HALF, M_HALF)]
            xt = x_ref[:, pl.ds((1 - my_x) * M_HALF, M_HALF)]

            mines = []
            x_rdmas = []
            for c in range(NC):
                dyc = dy_ref[:, pl.ds(my_y * F_HALF + c * CS, CS)]
                pt = lax.dot_general(xt, dyc, (((0,), (0,)), ((), ())),
                                     preferred_element_type=jnp.float32)
                send_x[:, c * CS:(c + 1) * CS] = pt.astype(jnp.bfloat16)
                if c == 1:
                    pl.semaphore_wait(bs, 1)
                    x_rdmas[0].start()
                rdma = pltpu.make_async_remote_copy(
                    src_ref=send_x.at[:, pl.ds(c * CS, CS)],
                    dst_ref=recv_x.at[:, pl.ds(c * CS, CS)],
                    send_sem=sx_sems.at[c], recv_sem=rx_sems.at[c],
                    device_id=(1 - my_x, my_y),
                    device_id_type=pl.DeviceIdType.MESH)
                if c != 0:
                    rdma.start()
                x_rdmas.append(rdma)
                mines.append(
                    lax.dot_general(xm, dyc, (((0,), (0,)), ((), ())),
                                    preferred_element_type=jnp.float32))

            y_rdmas = []
            for c in range(NC):
                x_rdmas[c].wait()
                q = (mines[c]
                     + recv_x[:, c * CS:(c + 1) * CS].astype(jnp.float32))
                send_y[:, c * CS:(c + 1) * CS] = q.astype(jnp.bfloat16)
                if c == 0:
                    pl.semaphore_wait(ysem, 1)
                rdma = pltpu.make_async_remote_copy(
                    src_ref=send_y.at[:, pl.ds(c * CS, CS)],
                    dst_ref=recv_y.at[:, pl.ds(c * CS, CS)],
                    send_sem=sy_sems.at[c], recv_sem=ry_sems.at[c],
                    device_id=(my_x, 1 - my_y),
                    device_id_type=pl.DeviceIdType.MESH)
                rdma.start()
                y_rdmas.append(rdma)
                out_ref[:, pl.ds(my_y * F_HALF + c * CS, CS)] = q

            for c in range(NC):
                y_rdmas[c].wait()
                out_ref[:, pl.ds((1 - my_y) * F_HALF + c * CS, CS)] = (
                    recv_y[:, c * CS:(c + 1) * CS].astype(jnp.float32))

        pl.run_scoped(inner, ysem=pltpu.SemaphoreType.REGULAR)

    return pl.pallas_call(
        body,
        out_shape=jax.ShapeDtypeStruct((M_HALF, F), jnp.float32),
        in_specs=[pl.BlockSpec(memory_space=pltpu.VMEM),
                  pl.BlockSpec(memory_space=pltpu.VMEM)],
        out_specs=pl.BlockSpec(memory_space=pltpu.VMEM),
        scratch_shapes=[
            pltpu.VMEM((M_HALF, F_HALF), jnp.bfloat16),
            pltpu.VMEM((M_HALF, F_HALF), jnp.bfloat16),
            pltpu.VMEM((M_HALF, F_HALF), jnp.bfloat16),
            pltpu.VMEM((M_HALF, F_HALF), jnp.bfloat16),
            pltpu.SemaphoreType.DMA((NC,)),
            pltpu.SemaphoreType.DMA((NC,)),
            pltpu.SemaphoreType.DMA((NC,)),
            pltpu.SemaphoreType.DMA((NC,)),
        ],
        compiler_params=pltpu.CompilerParams(collective_id=0),
    )(x, dy)
